# baseline (speedup 1.0000x reference)
import jax
import jax.numpy as jnp
from jax.experimental import pallas as pl
from jax.experimental.pallas import tpu as pltpu
from jax.experimental.pallas import tpu_sc as plsc

_L = 512
_D = 128
_BI = 32
_WIN = 128


def _sc_gather(w_i, w_j, seq):
    seq2d = seq.reshape(1, _L)
    row_ty = jax.ShapeDtypeStruct((_L, _D), jnp.float32)

    @pl.kernel(
        out_type=(row_ty, row_ty),
        mesh=plsc.VectorSubcoreMesh(core_axis_name="core",
                                    subcore_axis_name="subcore"),
    )
    def gather_kernel(wi_hbm, wj_hbm, idx_hbm, gi_hbm, gj_hbm):
        def body(i_vmem, oi_vmem, oj_vmem):
            pltpu.sync_copy(wi_hbm.at[i_vmem.at[0]], oi_vmem)
            pltpu.sync_copy(wj_hbm.at[i_vmem.at[0]], oj_vmem)

        pltpu.emit_pipeline(
            body,
            grid=(_L // _WIN,),
            in_specs=[pl.BlockSpec((1, _WIN), index_map=lambda i: (0, i))],
            out_specs=[pl.BlockSpec((_WIN, _D), index_map=lambda i: (i, 0)),
                       pl.BlockSpec((_WIN, _D), index_map=lambda i: (i, 0))],
            core_axis_name=("core", "subcore"),
            dimension_semantics=(pltpu.PARALLEL,),
        )(idx_hbm, gi_hbm, gj_hbm)

    return gather_kernel(w_i, w_j, seq2d)


def _edge_body(wrel_ref, gi_ref, gj_ref, x_ref, o_ref, r3_ref):
    @pl.when(pl.program_id(0) == 0)
    def _build_r3():
        k = jax.lax.broadcasted_iota(jnp.int32, (1024, 1), 0)
        ridx = jnp.clip(k - (_L - 1), -32, 32) + 32
        onehot = (jax.lax.broadcasted_iota(jnp.int32, (1024, 65), 1)
                  == ridx).astype(jnp.float32)
        r3_ref[...] = jax.lax.dot_general(
            onehot, wrel_ref[...], (((1,), (0,)), ((), ())),
            preferred_element_type=jnp.float32)

    i0 = pl.program_id(0) * _BI
    pj = gj_ref[...]
    for r in range(_BI):
        pi = gi_ref[pl.ds(i0 + r, 1), :]
        rel = r3_ref[pl.ds(_L - 1 - (i0 + r), _L), :]
        o_ref[r] = x_ref[r] + pi + pj + rel


def kernel(fasta_sequence, out, W_i, W_j, W_rel):
    seq = fasta_sequence.reshape(_L).astype(jnp.int32)

    g_i, g_j = _sc_gather(W_i, W_j, seq)

    x = out.reshape(_L, _L, _D)
    res = pl.pallas_call(
        _edge_body,
        grid=(_L // _BI,),
        in_specs=[
            pl.BlockSpec((65, _D), lambda i: (0, 0)),
            pl.BlockSpec((_L, _D), lambda i: (0, 0)),
            pl.BlockSpec((_L, _D), lambda i: (0, 0)),
            pl.BlockSpec((_BI, _L, _D), lambda i: (i, 0, 0)),
        ],
        out_specs=pl.BlockSpec((_BI, _L, _D), lambda i: (i, 0, 0)),
        out_shape=jax.ShapeDtypeStruct((_L, _L, _D), jnp.float32),
        scratch_shapes=[pltpu.VMEM((1024, _D), jnp.float32)],
    )(W_rel, g_i, g_j, x)
    return res.reshape(out.shape)
